# baseline (speedup 1.0000x reference)
import functools
import jax
import jax.numpy as jnp
from jax import lax
from jax.experimental import pallas as pl
from jax.experimental.pallas import tpu as pltpu
from jax.experimental.pallas import tpu_sc as plsc

E = 64
TOPK = 2
D = 2048
DFF = 1408
DSH = 2816
S = 2048

BM = 128
WU = 96
NPAD = WU * BM
BF = 128
KG = DFF // BF
KS = DSH // BF
TS = S // BM


def _router_body(x_ref, gw_ref, w0_ref, w1_ref, e0_ref, e1_ref):
    x = x_ref[...]
    logits = lax.dot_general(x, gw_ref[...], (((1,), (1,)), ((), ())),
                             preferred_element_type=jnp.float32)
    m = jnp.max(logits, axis=1, keepdims=True)
    p = jnp.exp(logits - m)
    z = jnp.sum(p, axis=1, keepdims=True)
    iota = lax.broadcasted_iota(jnp.int32, p.shape, 1)
    v0 = jnp.max(p, axis=1, keepdims=True)
    i0 = jnp.min(jnp.where(p == v0, iota, E), axis=1, keepdims=True)
    p2 = jnp.where(iota == i0, -1.0, p)
    v1 = jnp.max(p2, axis=1, keepdims=True)
    i1 = jnp.min(jnp.where(p2 == v1, iota, E), axis=1, keepdims=True)
    s0 = v0 / z
    s1 = v1 / z
    denom = s0 + s1 + 1e-20
    w0_ref[...] = s0 / denom
    w1_ref[...] = s1 / denom
    e0_ref[...] = i0
    e1_ref[...] = i1


def _run_router(x, gate_weight):
    return pl.pallas_call(
        _router_body,
        grid=(TS,),
        in_specs=[
            pl.BlockSpec((BM, D), lambda t: (t, 0)),
            pl.BlockSpec((E, D), lambda t: (0, 0)),
        ],
        out_specs=[
            pl.BlockSpec((BM, 1), lambda t: (t, 0)),
            pl.BlockSpec((BM, 1), lambda t: (t, 0)),
            pl.BlockSpec((BM, 1), lambda t: (t, 0)),
            pl.BlockSpec((BM, 1), lambda t: (t, 0)),
        ],
        out_shape=[
            jax.ShapeDtypeStruct((S, 1), jnp.float32),
            jax.ShapeDtypeStruct((S, 1), jnp.float32),
            jax.ShapeDtypeStruct((S, 1), jnp.int32),
            jax.ShapeDtypeStruct((S, 1), jnp.int32),
        ],
    )(x, gate_weight)


def _make_sc_gather(V, B, CH):
    info = plsc.get_sparse_core_info()
    NW = info.num_cores * info.num_subcores
    b_per_w = B // NW
    n_ch = b_per_w // CH
    mesh = plsc.VectorSubcoreMesh(core_axis_name="c", subcore_axis_name="s")

    @functools.partial(
        pl.kernel, mesh=mesh,
        out_type=jax.ShapeDtypeStruct((B, D), jnp.float32),
        scratch_types=[
            pltpu.VMEM((CH,), jnp.int32),
            pltpu.VMEM((CH,), jnp.int32),
            pltpu.VMEM((CH, D), jnp.float32),
            pltpu.VMEM((CH, D), jnp.float32),
            pltpu.SemaphoreType.DMA,
            pltpu.SemaphoreType.DMA,
        ],
    )
    def k(table_hbm, idx_hbm, out_hbm, idx0, idx1, rows0, rows1, sem0, sem1):
        wid = lax.axis_index("s") * info.num_cores + lax.axis_index("c")
        base = wid * b_per_w
        idx_v = (idx0, idx1)
        rows_v = (rows0, rows1)
        sems = (sem0, sem1)
        pltpu.sync_copy(idx_hbm.at[pl.ds(base, CH)], idx0)
        dma = pltpu.async_copy(table_hbm.at[idx0], rows0, sem0)
        for c in range(n_ch):
            b = c % 2
            nb = (c + 1) % 2
            if c + 1 < n_ch:
                off_n = base + (c + 1) * CH
                pltpu.sync_copy(idx_hbm.at[pl.ds(off_n, CH)], idx_v[nb])
                dma_n = pltpu.async_copy(table_hbm.at[idx_v[nb]], rows_v[nb],
                                         sems[nb])
            dma.wait()
            pltpu.sync_copy(rows_v[b], out_hbm.at[pl.ds(base + c * CH, CH)])
            if c + 1 < n_ch:
                dma = dma_n

    return k


def _group_body(ue_ref, ut_ref, us_ref, uen_ref, uf_ref,
                xs_ref, sw_ref, wg_ref, wuq_ref, wd_ref, y_ref):
    u = pl.program_id(0)
    k = pl.program_id(1)

    @pl.when(uen_ref[u] > 0)
    def _():
        x = xs_ref[...].astype(jnp.bfloat16)
        g = wg_ref[0].astype(jnp.bfloat16)
        up = wuq_ref[0].astype(jnp.bfloat16)
        hg = lax.dot_general(x, g, (((1,), (1,)), ((), ())),
                             preferred_element_type=jnp.float32)
        hu = lax.dot_general(x, up, (((1,), (1,)), ((), ())),
                             preferred_element_type=jnp.float32)
        h = hg * jax.nn.sigmoid(hg) * hu
        r = lax.broadcasted_iota(jnp.int32, (BM, 1), 0)
        valid = (r >= us_ref[u]) & (r < uen_ref[u])
        h = h * jnp.where(valid, sw_ref[...], 0.0)
        d = wd_ref[0].astype(jnp.bfloat16)
        part = lax.dot_general(h.astype(jnp.bfloat16), d,
                               (((1,), (1,)), ((), ())),
                               preferred_element_type=jnp.float32)
        init = (uf_ref[u] != 0) & (k == 0)

        @pl.when(init)
        def _():
            y_ref[...] = part

        @pl.when(jnp.logical_not(init))
        def _():
            y_ref[...] += part


def _run_grouped(xs, sw, w_gate, w_up, w_down, ue, ut, us, uen, uf):
    grid_spec = pltpu.PrefetchScalarGridSpec(
        num_scalar_prefetch=5,
        grid=(WU, KG),
        in_specs=[
            pl.BlockSpec((BM, D), lambda u, k, ue, ut, us, uen, uf: (ut[u], 0)),
            pl.BlockSpec((BM, 1), lambda u, k, ue, ut, us, uen, uf: (ut[u], 0)),
            pl.BlockSpec((1, BF, D),
                         lambda u, k, ue, ut, us, uen, uf: (ue[u], k, 0)),
            pl.BlockSpec((1, BF, D),
                         lambda u, k, ue, ut, us, uen, uf: (ue[u], k, 0)),
            pl.BlockSpec((1, D, BF),
                         lambda u, k, ue, ut, us, uen, uf: (ue[u], 0, k)),
        ],
        out_specs=pl.BlockSpec((BM, D), lambda u, k, ue, ut, us, uen, uf: (ut[u], 0)),
    )
    return pl.pallas_call(
        _group_body,
        grid_spec=grid_spec,
        out_shape=jax.ShapeDtypeStruct((NPAD, D), jnp.float32),
    )(ue, ut, us, uen, uf, xs, sw, w_gate, w_up, w_down)


def _shared_body(x_ref, g_ref, u_ref, d_ref, out_ref):
    k = pl.program_id(1)
    x = x_ref[...].astype(jnp.bfloat16)
    hg = lax.dot_general(x, g_ref[...].astype(jnp.bfloat16),
                         (((1,), (1,)), ((), ())),
                         preferred_element_type=jnp.float32)
    hu = lax.dot_general(x, u_ref[...].astype(jnp.bfloat16),
                         (((1,), (1,)), ((), ())),
                         preferred_element_type=jnp.float32)
    h = hg * jax.nn.sigmoid(hg) * hu
    part = lax.dot_general(h.astype(jnp.bfloat16),
                           d_ref[...].astype(jnp.bfloat16),
                           (((1,), (1,)), ((), ())),
                           preferred_element_type=jnp.float32)

    @pl.when(k == 0)
    def _():
        out_ref[...] = part

    @pl.when(k != 0)
    def _():
        out_ref[...] += part


def _run_shared(x, sh_gate, sh_up, sh_down):
    return pl.pallas_call(
        _shared_body,
        grid=(TS, KS),
        in_specs=[
            pl.BlockSpec((BM, D), lambda t, k: (t, 0)),
            pl.BlockSpec((BF, D), lambda t, k: (k, 0)),
            pl.BlockSpec((BF, D), lambda t, k: (k, 0)),
            pl.BlockSpec((D, BF), lambda t, k: (0, k)),
        ],
        out_specs=pl.BlockSpec((BM, D), lambda t, k: (t, 0)),
        out_shape=jax.ShapeDtypeStruct((S, D), jnp.float32),
    )(x, sh_gate, sh_up, sh_down)


def _final_body(ysh_ref, z0_ref, z1_ref, out_ref):
    out_ref[...] = ysh_ref[...] + z0_ref[...] + z1_ref[...]


def _run_final(ysh, z):
    return pl.pallas_call(
        _final_body,
        grid=(TS,),
        in_specs=[
            pl.BlockSpec((BM, D), lambda t: (t, 0)),
            pl.BlockSpec((BM, D), lambda t: (t, 0)),
            pl.BlockSpec((BM, D), lambda t: (TS + t, 0)),
        ],
        out_specs=pl.BlockSpec((BM, D), lambda t: (t, 0)),
        out_shape=jax.ShapeDtypeStruct((S, D), jnp.float32),
    )(ysh, z, z)


def kernel(hidden_states, gate_weight, w_gate, w_up, w_down,
           sh_gate, sh_up, sh_down):
    bsz, seq, h = hidden_states.shape
    x = hidden_states.reshape(-1, h)

    w0, w1, e0, e1 = _run_router(x, gate_weight)

    eflat = jnp.concatenate([e0, e1], axis=1).reshape(-1)
    wflat = jnp.concatenate([w0, w1], axis=1).reshape(-1)
    perm = jnp.argsort(eflat).astype(jnp.int32)
    sorted_eid = eflat[perm]
    sorted_tid = (perm // TOPK).astype(jnp.int32)
    sorted_w = wflat[perm]
    pos = jnp.zeros((S * TOPK,), jnp.int32).at[perm].set(
        jnp.arange(S * TOPK, dtype=jnp.int32))

    counts = jnp.zeros((E,), jnp.int32).at[eflat].add(1)
    offs = (jnp.cumsum(counts) - counts).astype(jnp.int32)
    ntile = (counts + BM - 1) // BM
    cumt = jnp.cumsum(ntile).astype(jnp.int32)
    ptile = (cumt - ntile).astype(jnp.int32)
    total_tiles = cumt[-1]

    rarr = jnp.arange(S * TOPK, dtype=jnp.int32)
    qpos = ptile[sorted_eid] * BM + (rarr - offs[sorted_eid])
    tidp = (jnp.arange(NPAD, dtype=jnp.int32) % S).at[qpos].set(sorted_tid)
    swp = jnp.zeros((NPAD,), jnp.float32).at[qpos].set(sorted_w)
    swp = swp.reshape(NPAD, 1)
    qslot = jnp.zeros((S * TOPK,), jnp.int32).at[perm].set(qpos)
    pp = qslot.reshape(S, TOPK).T.reshape(-1)

    uarr = jnp.arange(WU, dtype=jnp.int32)
    eu = jnp.minimum(jnp.searchsorted(cumt, uarr, side="right"),
                     E - 1).astype(jnp.int32)
    valid_u = uarr < total_tiles
    last_e = sorted_eid[-1]
    ue = jnp.where(valid_u, eu, last_e).astype(jnp.int32)
    ut = jnp.where(valid_u, uarr, total_tiles - 1).astype(jnp.int32)
    us = jnp.zeros((WU,), jnp.int32)
    uen = jnp.where(valid_u, BM, 0).astype(jnp.int32)
    uf = valid_u.astype(jnp.int32)

    ysh = _run_shared(x, sh_gate, sh_up, sh_down)

    xs = _make_sc_gather(S, NPAD, 24)(x, tidp)

    y = _run_grouped(xs, swp, w_gate, w_up, w_down, ue, ut, us, uen, uf)

    z = _make_sc_gather(NPAD, S * TOPK, 16)(y, pp)

    out = _run_final(ysh, z)
    return out.reshape(bsz, seq, h)


# speedup vs baseline: 1.0739x; 1.0739x over previous
import functools
import jax
import jax.numpy as jnp
from jax import lax
from jax.experimental import pallas as pl
from jax.experimental.pallas import tpu as pltpu
from jax.experimental.pallas import tpu_sc as plsc

E = 64
TOPK = 2
D = 2048
DFF = 1408
DSH = 2816
S = 2048

BM = 128
WU = 96
NPAD = WU * BM
BF = 128
KG = DFF // BF
KS = DSH // BF
TS = S // BM


def _router_body(x_ref, gw_ref, w0_ref, w1_ref, e0_ref, e1_ref):
    x = x_ref[...]
    logits = lax.dot_general(x, gw_ref[...], (((1,), (1,)), ((), ())),
                             preferred_element_type=jnp.float32)
    m = jnp.max(logits, axis=1, keepdims=True)
    p = jnp.exp(logits - m)
    z = jnp.sum(p, axis=1, keepdims=True)
    iota = lax.broadcasted_iota(jnp.int32, p.shape, 1)
    v0 = jnp.max(p, axis=1, keepdims=True)
    i0 = jnp.min(jnp.where(p == v0, iota, E), axis=1, keepdims=True)
    p2 = jnp.where(iota == i0, -1.0, p)
    v1 = jnp.max(p2, axis=1, keepdims=True)
    i1 = jnp.min(jnp.where(p2 == v1, iota, E), axis=1, keepdims=True)
    s0 = v0 / z
    s1 = v1 / z
    denom = s0 + s1 + 1e-20
    w0_ref[...] = s0 / denom
    w1_ref[...] = s1 / denom
    e0_ref[...] = i0
    e1_ref[...] = i1


def _run_router(x, gate_weight):
    return pl.pallas_call(
        _router_body,
        grid=(TS,),
        in_specs=[
            pl.BlockSpec((BM, D), lambda t: (t, 0)),
            pl.BlockSpec((E, D), lambda t: (0, 0)),
        ],
        out_specs=[
            pl.BlockSpec((BM, 1), lambda t: (t, 0)),
            pl.BlockSpec((BM, 1), lambda t: (t, 0)),
            pl.BlockSpec((BM, 1), lambda t: (t, 0)),
            pl.BlockSpec((BM, 1), lambda t: (t, 0)),
        ],
        out_shape=[
            jax.ShapeDtypeStruct((S, 1), jnp.float32),
            jax.ShapeDtypeStruct((S, 1), jnp.float32),
            jax.ShapeDtypeStruct((S, 1), jnp.int32),
            jax.ShapeDtypeStruct((S, 1), jnp.int32),
        ],
    )(x, gate_weight)


def _make_sc_gather(V, B, CH):
    info = plsc.get_sparse_core_info()
    NW = info.num_cores * info.num_subcores
    b_per_w = B // NW
    n_ch = b_per_w // CH
    mesh = plsc.VectorSubcoreMesh(core_axis_name="c", subcore_axis_name="s")

    @functools.partial(
        pl.kernel, mesh=mesh,
        out_type=jax.ShapeDtypeStruct((B, D), jnp.float32),
        scratch_types=[
            pltpu.VMEM((CH,), jnp.int32),
            pltpu.VMEM((CH,), jnp.int32),
            pltpu.VMEM((CH, D), jnp.float32),
            pltpu.VMEM((CH, D), jnp.float32),
            pltpu.SemaphoreType.DMA,
            pltpu.SemaphoreType.DMA,
        ],
    )
    def k(table_hbm, idx_hbm, out_hbm, idx0, idx1, rows0, rows1, sem0, sem1):
        wid = lax.axis_index("s") * info.num_cores + lax.axis_index("c")
        base = wid * b_per_w
        idx_v = (idx0, idx1)
        rows_v = (rows0, rows1)
        sems = (sem0, sem1)
        pltpu.sync_copy(idx_hbm.at[pl.ds(base, CH)], idx0)
        dma = pltpu.async_copy(table_hbm.at[idx0], rows0, sem0)
        for c in range(n_ch):
            b = c % 2
            nb = (c + 1) % 2
            if c + 1 < n_ch:
                off_n = base + (c + 1) * CH
                pltpu.sync_copy(idx_hbm.at[pl.ds(off_n, CH)], idx_v[nb])
                dma_n = pltpu.async_copy(table_hbm.at[idx_v[nb]], rows_v[nb],
                                         sems[nb])
            dma.wait()
            pltpu.sync_copy(rows_v[b], out_hbm.at[pl.ds(base + c * CH, CH)])
            if c + 1 < n_ch:
                dma = dma_n

    return k


def _group_body(ue_ref, ut_ref, us_ref, uen_ref, uf_ref,
                xs_ref, sw_ref, wg0_ref, wg1_ref, wu0_ref, wu1_ref,
                wd0_ref, wd1_ref, y_ref):
    u = pl.program_id(0)
    k = pl.program_id(1)

    @pl.when(uen_ref[u] > 0)
    def _():
        x = xs_ref[...].astype(jnp.bfloat16)
        r = lax.broadcasted_iota(jnp.int32, (BM, 1), 0)
        valid = (r >= us_ref[u]) & (r < uen_ref[u])
        scale = jnp.where(valid, sw_ref[...], 0.0)

        def chunk(wg_ref, wuq_ref, wd_ref):
            g = wg_ref[0].astype(jnp.bfloat16)
            up = wuq_ref[0].astype(jnp.bfloat16)
            hg = lax.dot_general(x, g, (((1,), (1,)), ((), ())),
                                 preferred_element_type=jnp.float32)
            hu = lax.dot_general(x, up, (((1,), (1,)), ((), ())),
                                 preferred_element_type=jnp.float32)
            h = (hg * jax.nn.sigmoid(hg) * hu) * scale
            d = wd_ref[0].astype(jnp.bfloat16)
            return lax.dot_general(h.astype(jnp.bfloat16), d,
                                   (((1,), (1,)), ((), ())),
                                   preferred_element_type=jnp.float32)

        part = chunk(wg0_ref, wu0_ref, wd0_ref)
        init = (uf_ref[u] != 0) & (k == 0)

        @pl.when(init)
        def _():
            y_ref[...] = part

        @pl.when(jnp.logical_not(init))
        def _():
            y_ref[...] += part

        @pl.when(2 * k + 1 < KG)
        def _():
            y_ref[...] += chunk(wg1_ref, wu1_ref, wd1_ref)


def _c1(k):
    return jnp.where(2 * k + 1 < KG, 2 * k + 1, KG - 2)


def _run_grouped(xs, sw, w_gate, w_up, w_down, ue, ut, us, uen, uf):
    kp = (KG + 1) // 2
    grid_spec = pltpu.PrefetchScalarGridSpec(
        num_scalar_prefetch=5,
        grid=(WU, kp),
        in_specs=[
            pl.BlockSpec((BM, D), lambda u, k, ue, ut, us, uen, uf: (ut[u], 0)),
            pl.BlockSpec((BM, 1), lambda u, k, ue, ut, us, uen, uf: (ut[u], 0)),
            pl.BlockSpec((1, BF, D),
                         lambda u, k, ue, ut, us, uen, uf: (ue[u], 2 * k, 0)),
            pl.BlockSpec((1, BF, D),
                         lambda u, k, ue, ut, us, uen, uf: (ue[u], _c1(k), 0)),
            pl.BlockSpec((1, BF, D),
                         lambda u, k, ue, ut, us, uen, uf: (ue[u], 2 * k, 0)),
            pl.BlockSpec((1, BF, D),
                         lambda u, k, ue, ut, us, uen, uf: (ue[u], _c1(k), 0)),
            pl.BlockSpec((1, D, BF),
                         lambda u, k, ue, ut, us, uen, uf: (ue[u], 0, 2 * k)),
            pl.BlockSpec((1, D, BF),
                         lambda u, k, ue, ut, us, uen, uf: (ue[u], 0, _c1(k))),
        ],
        out_specs=pl.BlockSpec((BM, D), lambda u, k, ue, ut, us, uen, uf: (ut[u], 0)),
    )
    return pl.pallas_call(
        _group_body,
        grid_spec=grid_spec,
        out_shape=jax.ShapeDtypeStruct((NPAD, D), jnp.float32),
    )(ue, ut, us, uen, uf, xs, sw, w_gate, w_gate, w_up, w_up,
      w_down, w_down)


def _shared_body(x_ref, g_ref, u_ref, d_ref, out_ref):
    k = pl.program_id(1)
    x = x_ref[...].astype(jnp.bfloat16)
    hg = lax.dot_general(x, g_ref[...].astype(jnp.bfloat16),
                         (((1,), (1,)), ((), ())),
                         preferred_element_type=jnp.float32)
    hu = lax.dot_general(x, u_ref[...].astype(jnp.bfloat16),
                         (((1,), (1,)), ((), ())),
                         preferred_element_type=jnp.float32)
    h = hg * jax.nn.sigmoid(hg) * hu
    part = lax.dot_general(h.astype(jnp.bfloat16),
                           d_ref[...].astype(jnp.bfloat16),
                           (((1,), (1,)), ((), ())),
                           preferred_element_type=jnp.float32)

    @pl.when(k == 0)
    def _():
        out_ref[...] = part

    @pl.when(k != 0)
    def _():
        out_ref[...] += part


def _run_shared(x, sh_gate, sh_up, sh_down):
    return pl.pallas_call(
        _shared_body,
        grid=(TS, KS),
        in_specs=[
            pl.BlockSpec((BM, D), lambda t, k: (t, 0)),
            pl.BlockSpec((BF, D), lambda t, k: (k, 0)),
            pl.BlockSpec((BF, D), lambda t, k: (k, 0)),
            pl.BlockSpec((D, BF), lambda t, k: (0, k)),
        ],
        out_specs=pl.BlockSpec((BM, D), lambda t, k: (t, 0)),
        out_shape=jax.ShapeDtypeStruct((S, D), jnp.float32),
    )(x, sh_gate, sh_up, sh_down)


def _final_body(ysh_ref, z0_ref, z1_ref, out_ref):
    out_ref[...] = ysh_ref[...] + z0_ref[...] + z1_ref[...]


def _run_final(ysh, z):
    return pl.pallas_call(
        _final_body,
        grid=(TS,),
        in_specs=[
            pl.BlockSpec((BM, D), lambda t: (t, 0)),
            pl.BlockSpec((BM, D), lambda t: (t, 0)),
            pl.BlockSpec((BM, D), lambda t: (TS + t, 0)),
        ],
        out_specs=pl.BlockSpec((BM, D), lambda t: (t, 0)),
        out_shape=jax.ShapeDtypeStruct((S, D), jnp.float32),
    )(ysh, z, z)


def kernel(hidden_states, gate_weight, w_gate, w_up, w_down,
           sh_gate, sh_up, sh_down):
    bsz, seq, h = hidden_states.shape
    x = hidden_states.reshape(-1, h)

    w0, w1, e0, e1 = _run_router(x, gate_weight)

    eflat = jnp.concatenate([e0, e1], axis=1).reshape(-1)
    wflat = jnp.concatenate([w0, w1], axis=1).reshape(-1)
    perm = jnp.argsort(eflat).astype(jnp.int32)
    sorted_eid = eflat[perm]
    sorted_tid = (perm // TOPK).astype(jnp.int32)
    sorted_w = wflat[perm]
    pos = jnp.zeros((S * TOPK,), jnp.int32).at[perm].set(
        jnp.arange(S * TOPK, dtype=jnp.int32))

    counts = jnp.zeros((E,), jnp.int32).at[eflat].add(1)
    offs = (jnp.cumsum(counts) - counts).astype(jnp.int32)
    ntile = (counts + BM - 1) // BM
    cumt = jnp.cumsum(ntile).astype(jnp.int32)
    ptile = (cumt - ntile).astype(jnp.int32)
    total_tiles = cumt[-1]

    rarr = jnp.arange(S * TOPK, dtype=jnp.int32)
    qpos = ptile[sorted_eid] * BM + (rarr - offs[sorted_eid])
    tidp = (jnp.arange(NPAD, dtype=jnp.int32) % S).at[qpos].set(sorted_tid)
    swp = jnp.zeros((NPAD,), jnp.float32).at[qpos].set(sorted_w)
    swp = swp.reshape(NPAD, 1)
    qslot = jnp.zeros((S * TOPK,), jnp.int32).at[perm].set(qpos)
    pp = qslot.reshape(S, TOPK).T.reshape(-1)

    uarr = jnp.arange(WU, dtype=jnp.int32)
    eu = jnp.minimum(jnp.searchsorted(cumt, uarr, side="right"),
                     E - 1).astype(jnp.int32)
    valid_u = uarr < total_tiles
    last_e = sorted_eid[-1]
    ue = jnp.where(valid_u, eu, last_e).astype(jnp.int32)
    ut = jnp.where(valid_u, uarr, total_tiles - 1).astype(jnp.int32)
    us = jnp.zeros((WU,), jnp.int32)
    uen = jnp.where(valid_u, BM, 0).astype(jnp.int32)
    uf = valid_u.astype(jnp.int32)

    ysh = _run_shared(x, sh_gate, sh_up, sh_down)

    xs = _make_sc_gather(S, NPAD, 24)(x, tidp)

    y = _run_grouped(xs, swp, w_gate, w_up, w_down, ue, ut, us, uen, uf)

    z = _make_sc_gather(NPAD, S * TOPK, 16)(y, pp)

    out = _run_final(ysh, z)
    return out.reshape(bsz, seq, h)


# speedup vs baseline: 1.2221x; 1.1380x over previous
import functools
import jax
import jax.numpy as jnp
from jax import lax
from jax.experimental import pallas as pl
from jax.experimental.pallas import tpu as pltpu
from jax.experimental.pallas import tpu_sc as plsc

E = 64
TOPK = 2
D = 2048
DFF = 1408
DSH = 2816
S = 2048

BM = 128
WU = 96
NPAD = WU * BM
BF = 128
KG = DFF // BF
KS = DSH // BF
TS = S // BM


def _router_body(x_ref, gw_ref, w0_ref, w1_ref, e0_ref, e1_ref):
    x = x_ref[...]
    logits = lax.dot_general(x, gw_ref[...], (((1,), (1,)), ((), ())),
                             preferred_element_type=jnp.float32)
    m = jnp.max(logits, axis=1, keepdims=True)
    p = jnp.exp(logits - m)
    z = jnp.sum(p, axis=1, keepdims=True)
    iota = lax.broadcasted_iota(jnp.int32, p.shape, 1)
    v0 = jnp.max(p, axis=1, keepdims=True)
    i0 = jnp.min(jnp.where(p == v0, iota, E), axis=1, keepdims=True)
    p2 = jnp.where(iota == i0, -1.0, p)
    v1 = jnp.max(p2, axis=1, keepdims=True)
    i1 = jnp.min(jnp.where(p2 == v1, iota, E), axis=1, keepdims=True)
    s0 = v0 / z
    s1 = v1 / z
    denom = s0 + s1 + 1e-20
    w0_ref[...] = s0 / denom
    w1_ref[...] = s1 / denom
    e0_ref[...] = i0
    e1_ref[...] = i1


def _run_router(x, gate_weight):
    return pl.pallas_call(
        _router_body,
        grid=(TS,),
        in_specs=[
            pl.BlockSpec((BM, D), lambda t: (t, 0)),
            pl.BlockSpec((E, D), lambda t: (0, 0)),
        ],
        out_specs=[
            pl.BlockSpec((BM, 1), lambda t: (t, 0)),
            pl.BlockSpec((BM, 1), lambda t: (t, 0)),
            pl.BlockSpec((BM, 1), lambda t: (t, 0)),
            pl.BlockSpec((BM, 1), lambda t: (t, 0)),
        ],
        out_shape=[
            jax.ShapeDtypeStruct((S, 1), jnp.float32),
            jax.ShapeDtypeStruct((S, 1), jnp.float32),
            jax.ShapeDtypeStruct((S, 1), jnp.int32),
            jax.ShapeDtypeStruct((S, 1), jnp.int32),
        ],
    )(x, gate_weight)


def _make_sc_gather(V, B, CH):
    info = plsc.get_sparse_core_info()
    NW = info.num_cores * info.num_subcores
    b_per_w = B // NW
    n_ch = b_per_w // CH
    mesh = plsc.VectorSubcoreMesh(core_axis_name="c", subcore_axis_name="s")

    @functools.partial(
        pl.kernel, mesh=mesh,
        out_type=jax.ShapeDtypeStruct((B, D), jnp.float32),
        scratch_types=[
            pltpu.VMEM((CH,), jnp.int32),
            pltpu.VMEM((CH,), jnp.int32),
            pltpu.VMEM((CH, D), jnp.float32),
            pltpu.VMEM((CH, D), jnp.float32),
            pltpu.SemaphoreType.DMA,
            pltpu.SemaphoreType.DMA,
        ],
    )
    def k(table_hbm, idx_hbm, out_hbm, idx0, idx1, rows0, rows1, sem0, sem1):
        wid = lax.axis_index("s") * info.num_cores + lax.axis_index("c")
        base = wid * b_per_w
        idx_v = (idx0, idx1)
        rows_v = (rows0, rows1)
        sems = (sem0, sem1)
        pltpu.sync_copy(idx_hbm.at[pl.ds(base, CH)], idx0)
        dma = pltpu.async_copy(table_hbm.at[idx0], rows0, sem0)
        for c in range(n_ch):
            b = c % 2
            nb = (c + 1) % 2
            if c + 1 < n_ch:
                off_n = base + (c + 1) * CH
                pltpu.sync_copy(idx_hbm.at[pl.ds(off_n, CH)], idx_v[nb])
                dma_n = pltpu.async_copy(table_hbm.at[idx_v[nb]], rows_v[nb],
                                         sems[nb])
            dma.wait()
            pltpu.sync_copy(rows_v[b], out_hbm.at[pl.ds(base + c * CH, CH)])
            if c + 1 < n_ch:
                dma = dma_n

    return k


NCH = 4


def _cj(k, j, ktot):
    c = NCH * k + j
    return jnp.where(c < ktot, c, jnp.maximum(c - NCH, 0))


def _group_body(ue_ref, ut_ref, us_ref, uen_ref, uf_ref,
                xs_ref, sw_ref, *rest):
    wrefs = rest[:3 * NCH]
    y_ref = rest[3 * NCH]
    u = pl.program_id(0)
    k = pl.program_id(1)

    @pl.when(uen_ref[u] > 0)
    def _():
        x = xs_ref[...].astype(jnp.bfloat16)
        r = lax.broadcasted_iota(jnp.int32, (BM, 1), 0)
        valid = (r >= us_ref[u]) & (r < uen_ref[u])
        scale = jnp.where(valid, sw_ref[...], 0.0)

        def chunk(j):
            g = wrefs[3 * j][0].astype(jnp.bfloat16)
            up = wrefs[3 * j + 1][0].astype(jnp.bfloat16)
            hg = lax.dot_general(x, g, (((1,), (1,)), ((), ())),
                                 preferred_element_type=jnp.float32)
            hu = lax.dot_general(x, up, (((1,), (1,)), ((), ())),
                                 preferred_element_type=jnp.float32)
            h = (hg * jax.nn.sigmoid(hg) * hu) * scale
            d = wrefs[3 * j + 2][0].astype(jnp.bfloat16)
            return lax.dot_general(h.astype(jnp.bfloat16), d,
                                   (((1,), (1,)), ((), ())),
                                   preferred_element_type=jnp.float32)

        part = chunk(0)
        init = (uf_ref[u] != 0) & (k == 0)

        @pl.when(init)
        def _():
            y_ref[...] = part

        @pl.when(jnp.logical_not(init))
        def _():
            y_ref[...] += part

        for j in range(1, NCH):
            @pl.when(NCH * k + j < KG)
            def _(j=j):
                y_ref[...] += chunk(j)


def _gw_spec(j):
    return pl.BlockSpec(
        (1, BF, D),
        lambda u, k, ue, ut, us, uen, uf: (ue[u], _cj(k, j, KG), 0))


def _gd_spec(j):
    return pl.BlockSpec(
        (1, D, BF),
        lambda u, k, ue, ut, us, uen, uf: (ue[u], 0, _cj(k, j, KG)))


def _run_grouped(xs, sw, w_gate, w_up, w_down, ue, ut, us, uen, uf):
    kp = -(-KG // NCH)
    wspecs = []
    wargs = []
    for j in range(NCH):
        wspecs += [_gw_spec(j), _gw_spec(j), _gd_spec(j)]
        wargs += [w_gate, w_up, w_down]
    grid_spec = pltpu.PrefetchScalarGridSpec(
        num_scalar_prefetch=5,
        grid=(WU, kp),
        in_specs=[
            pl.BlockSpec((BM, D), lambda u, k, ue, ut, us, uen, uf: (ut[u], 0)),
            pl.BlockSpec((BM, 1), lambda u, k, ue, ut, us, uen, uf: (ut[u], 0)),
        ] + wspecs,
        out_specs=pl.BlockSpec((BM, D), lambda u, k, ue, ut, us, uen, uf: (ut[u], 0)),
    )
    return pl.pallas_call(
        _group_body,
        grid_spec=grid_spec,
        out_shape=jax.ShapeDtypeStruct((NPAD, D), jnp.float32),
    )(ue, ut, us, uen, uf, xs, sw, *wargs)


def _shared_body(x_ref, *rest):
    wrefs = rest[:3 * NCH]
    out_ref = rest[3 * NCH]
    k = pl.program_id(1)
    x = x_ref[...].astype(jnp.bfloat16)

    def chunk(j):
        g = wrefs[3 * j][...].astype(jnp.bfloat16)
        up = wrefs[3 * j + 1][...].astype(jnp.bfloat16)
        hg = lax.dot_general(x, g, (((1,), (1,)), ((), ())),
                             preferred_element_type=jnp.float32)
        hu = lax.dot_general(x, up, (((1,), (1,)), ((), ())),
                             preferred_element_type=jnp.float32)
        h = hg * jax.nn.sigmoid(hg) * hu
        d = wrefs[3 * j + 2][...].astype(jnp.bfloat16)
        return lax.dot_general(h.astype(jnp.bfloat16), d,
                               (((1,), (1,)), ((), ())),
                               preferred_element_type=jnp.float32)

    part = chunk(0)

    @pl.when(k == 0)
    def _():
        out_ref[...] = part

    @pl.when(k != 0)
    def _():
        out_ref[...] += part

    for j in range(1, NCH):
        @pl.when(NCH * k + j < KS)
        def _(j=j):
            out_ref[...] += chunk(j)


def _sw_spec(j):
    return pl.BlockSpec((BF, D), lambda t, k: (_cj(k, j, KS), 0))


def _sd_spec(j):
    return pl.BlockSpec((D, BF), lambda t, k: (0, _cj(k, j, KS)))


def _run_shared(x, sh_gate, sh_up, sh_down):
    kp = -(-KS // NCH)
    wspecs = []
    wargs = []
    for j in range(NCH):
        wspecs += [_sw_spec(j), _sw_spec(j), _sd_spec(j)]
        wargs += [sh_gate, sh_up, sh_down]
    return pl.pallas_call(
        _shared_body,
        grid=(TS, kp),
        in_specs=[pl.BlockSpec((BM, D), lambda t, k: (t, 0))] + wspecs,
        out_specs=pl.BlockSpec((BM, D), lambda t, k: (t, 0)),
        out_shape=jax.ShapeDtypeStruct((S, D), jnp.float32),
    )(x, *wargs)


def _final_body(ysh_ref, z0_ref, z1_ref, out_ref):
    out_ref[...] = ysh_ref[...] + z0_ref[...] + z1_ref[...]


def _run_final(ysh, z):
    return pl.pallas_call(
        _final_body,
        grid=(TS,),
        in_specs=[
            pl.BlockSpec((BM, D), lambda t: (t, 0)),
            pl.BlockSpec((BM, D), lambda t: (t, 0)),
            pl.BlockSpec((BM, D), lambda t: (TS + t, 0)),
        ],
        out_specs=pl.BlockSpec((BM, D), lambda t: (t, 0)),
        out_shape=jax.ShapeDtypeStruct((S, D), jnp.float32),
    )(ysh, z, z)


def kernel(hidden_states, gate_weight, w_gate, w_up, w_down,
           sh_gate, sh_up, sh_down):
    bsz, seq, h = hidden_states.shape
    x = hidden_states.reshape(-1, h)

    w0, w1, e0, e1 = _run_router(x, gate_weight)

    eflat = jnp.concatenate([e0, e1], axis=1).reshape(-1)
    wflat = jnp.concatenate([w0, w1], axis=1).reshape(-1)
    perm = jnp.argsort(eflat).astype(jnp.int32)
    sorted_eid = eflat[perm]
    sorted_tid = (perm // TOPK).astype(jnp.int32)
    sorted_w = wflat[perm]
    pos = jnp.zeros((S * TOPK,), jnp.int32).at[perm].set(
        jnp.arange(S * TOPK, dtype=jnp.int32))

    counts = jnp.zeros((E,), jnp.int32).at[eflat].add(1)
    offs = (jnp.cumsum(counts) - counts).astype(jnp.int32)
    ntile = (counts + BM - 1) // BM
    cumt = jnp.cumsum(ntile).astype(jnp.int32)
    ptile = (cumt - ntile).astype(jnp.int32)
    total_tiles = cumt[-1]

    rarr = jnp.arange(S * TOPK, dtype=jnp.int32)
    qpos = ptile[sorted_eid] * BM + (rarr - offs[sorted_eid])
    tidp = (jnp.arange(NPAD, dtype=jnp.int32) % S).at[qpos].set(sorted_tid)
    swp = jnp.zeros((NPAD,), jnp.float32).at[qpos].set(sorted_w)
    swp = swp.reshape(NPAD, 1)
    qslot = jnp.zeros((S * TOPK,), jnp.int32).at[perm].set(qpos)
    pp = qslot.reshape(S, TOPK).T.reshape(-1)

    uarr = jnp.arange(WU, dtype=jnp.int32)
    eu = jnp.minimum(jnp.searchsorted(cumt, uarr, side="right"),
                     E - 1).astype(jnp.int32)
    valid_u = uarr < total_tiles
    last_e = sorted_eid[-1]
    ue = jnp.where(valid_u, eu, last_e).astype(jnp.int32)
    ut = jnp.where(valid_u, uarr, total_tiles - 1).astype(jnp.int32)
    us = jnp.zeros((WU,), jnp.int32)
    uen = jnp.where(valid_u, BM, 0).astype(jnp.int32)
    uf = valid_u.astype(jnp.int32)

    ysh = _run_shared(x, sh_gate, sh_up, sh_down)

    xs = _make_sc_gather(S, NPAD, 24)(x, tidp)

    y = _run_grouped(xs, swp, w_gate, w_up, w_down, ue, ut, us, uen, uf)

    z = _make_sc_gather(NPAD, S * TOPK, 16)(y, pp)

    out = _run_final(ysh, z)
    return out.reshape(bsz, seq, h)


# speedup vs baseline: 1.2596x; 1.0306x over previous
import functools
import jax
import jax.numpy as jnp
from jax import lax
from jax.experimental import pallas as pl
from jax.experimental.pallas import tpu as pltpu
from jax.experimental.pallas import tpu_sc as plsc

E = 64
TOPK = 2
D = 2048
DFF = 1408
DSH = 2816
S = 2048

BM = 128
WU = 96
NPAD = WU * BM
BF = 128
KG = DFF // BF
KS = DSH // BF
TS = S // BM


def _router_body(x_ref, gw_ref, w0_ref, w1_ref, e0_ref, e1_ref):
    x = x_ref[...]
    logits = lax.dot_general(x, gw_ref[...], (((1,), (1,)), ((), ())),
                             preferred_element_type=jnp.float32)
    m = jnp.max(logits, axis=1, keepdims=True)
    p = jnp.exp(logits - m)
    z = jnp.sum(p, axis=1, keepdims=True)
    iota = lax.broadcasted_iota(jnp.int32, p.shape, 1)
    v0 = jnp.max(p, axis=1, keepdims=True)
    i0 = jnp.min(jnp.where(p == v0, iota, E), axis=1, keepdims=True)
    p2 = jnp.where(iota == i0, -1.0, p)
    v1 = jnp.max(p2, axis=1, keepdims=True)
    i1 = jnp.min(jnp.where(p2 == v1, iota, E), axis=1, keepdims=True)
    s0 = v0 / z
    s1 = v1 / z
    denom = s0 + s1 + 1e-20
    w0_ref[...] = s0 / denom
    w1_ref[...] = s1 / denom
    e0_ref[...] = i0
    e1_ref[...] = i1


def _run_router(x, gate_weight):
    return pl.pallas_call(
        _router_body,
        grid=(TS,),
        in_specs=[
            pl.BlockSpec((BM, D), lambda t: (t, 0)),
            pl.BlockSpec((E, D), lambda t: (0, 0)),
        ],
        out_specs=[
            pl.BlockSpec((BM, 1), lambda t: (t, 0)),
            pl.BlockSpec((BM, 1), lambda t: (t, 0)),
            pl.BlockSpec((BM, 1), lambda t: (t, 0)),
            pl.BlockSpec((BM, 1), lambda t: (t, 0)),
        ],
        out_shape=[
            jax.ShapeDtypeStruct((S, 1), jnp.float32),
            jax.ShapeDtypeStruct((S, 1), jnp.float32),
            jax.ShapeDtypeStruct((S, 1), jnp.int32),
            jax.ShapeDtypeStruct((S, 1), jnp.int32),
        ],
    )(x, gate_weight)


def _make_sc_gather(V, B, CH):
    info = plsc.get_sparse_core_info()
    NW = info.num_cores * info.num_subcores
    b_per_w = B // NW
    n_ch = b_per_w // CH
    mesh = plsc.VectorSubcoreMesh(core_axis_name="c", subcore_axis_name="s")

    @functools.partial(
        pl.kernel, mesh=mesh,
        out_type=jax.ShapeDtypeStruct((B, D), jnp.float32),
        scratch_types=[
            pltpu.VMEM((CH,), jnp.int32),
            pltpu.VMEM((CH,), jnp.int32),
            pltpu.VMEM((CH, D), jnp.float32),
            pltpu.VMEM((CH, D), jnp.float32),
            pltpu.SemaphoreType.DMA,
            pltpu.SemaphoreType.DMA,
        ],
    )
    def k(table_hbm, idx_hbm, out_hbm, idx0, idx1, rows0, rows1, sem0, sem1):
        wid = lax.axis_index("s") * info.num_cores + lax.axis_index("c")
        base = wid * b_per_w
        idx_v = (idx0, idx1)
        rows_v = (rows0, rows1)
        sems = (sem0, sem1)
        pltpu.sync_copy(idx_hbm.at[pl.ds(base, CH)], idx0)
        dma = pltpu.async_copy(table_hbm.at[idx0], rows0, sem0)
        for c in range(n_ch):
            b = c % 2
            nb = (c + 1) % 2
            if c + 1 < n_ch:
                off_n = base + (c + 1) * CH
                pltpu.sync_copy(idx_hbm.at[pl.ds(off_n, CH)], idx_v[nb])
                dma_n = pltpu.async_copy(table_hbm.at[idx_v[nb]], rows_v[nb],
                                         sems[nb])
            dma.wait()
            pltpu.sync_copy(rows_v[b], out_hbm.at[pl.ds(base + c * CH, CH)])
            if c + 1 < n_ch:
                dma = dma_n

    return k


NCH = 6


def _cj(k, j, ktot):
    c = NCH * k + j
    return jnp.where(c < ktot, c, jnp.maximum(c - NCH, 0))


def _group_body(ue_ref, ut_ref, us_ref, uen_ref, uf_ref,
                xs_ref, sw_ref, *rest):
    wrefs = rest[:3 * NCH]
    y_ref = rest[3 * NCH]
    u = pl.program_id(0)
    k = pl.program_id(1)

    @pl.when(uen_ref[u] > 0)
    def _():
        x = xs_ref[...].astype(jnp.bfloat16)
        r = lax.broadcasted_iota(jnp.int32, (BM, 1), 0)
        valid = (r >= us_ref[u]) & (r < uen_ref[u])
        scale = jnp.where(valid, sw_ref[...], 0.0)

        def chunk(j):
            g = wrefs[3 * j][0].astype(jnp.bfloat16)
            up = wrefs[3 * j + 1][0].astype(jnp.bfloat16)
            hg = lax.dot_general(x, g, (((1,), (1,)), ((), ())),
                                 preferred_element_type=jnp.float32)
            hu = lax.dot_general(x, up, (((1,), (1,)), ((), ())),
                                 preferred_element_type=jnp.float32)
            h = (hg * jax.nn.sigmoid(hg) * hu) * scale
            d = wrefs[3 * j + 2][0].astype(jnp.bfloat16)
            return lax.dot_general(h.astype(jnp.bfloat16), d,
                                   (((1,), (1,)), ((), ())),
                                   preferred_element_type=jnp.float32)

        part = chunk(0)
        init = (uf_ref[u] != 0) & (k == 0)

        @pl.when(init)
        def _():
            y_ref[...] = part

        @pl.when(jnp.logical_not(init))
        def _():
            y_ref[...] += part

        for j in range(1, NCH):
            @pl.when(NCH * k + j < KG)
            def _(j=j):
                y_ref[...] += chunk(j)


def _gw_spec(j):
    return pl.BlockSpec(
        (1, BF, D),
        lambda u, k, ue, ut, us, uen, uf: (ue[u], _cj(k, j, KG), 0))


def _gd_spec(j):
    return pl.BlockSpec(
        (1, D, BF),
        lambda u, k, ue, ut, us, uen, uf: (ue[u], 0, _cj(k, j, KG)))


def _run_grouped(xs, sw, w_gate, w_up, w_down, ue, ut, us, uen, uf):
    kp = -(-KG // NCH)
    wspecs = []
    wargs = []
    for j in range(NCH):
        wspecs += [_gw_spec(j), _gw_spec(j), _gd_spec(j)]
        wargs += [w_gate, w_up, w_down]
    grid_spec = pltpu.PrefetchScalarGridSpec(
        num_scalar_prefetch=5,
        grid=(WU, kp),
        in_specs=[
            pl.BlockSpec((BM, D), lambda u, k, ue, ut, us, uen, uf: (ut[u], 0)),
            pl.BlockSpec((BM, 1), lambda u, k, ue, ut, us, uen, uf: (ut[u], 0)),
        ] + wspecs,
        out_specs=pl.BlockSpec((BM, D), lambda u, k, ue, ut, us, uen, uf: (ut[u], 0)),
    )
    return pl.pallas_call(
        _group_body,
        grid_spec=grid_spec,
        out_shape=jax.ShapeDtypeStruct((NPAD, D), jnp.float32),
    )(ue, ut, us, uen, uf, xs, sw, *wargs)


def _shared_body(x_ref, *rest):
    wrefs = rest[:3 * NCH]
    out_ref = rest[3 * NCH]
    k = pl.program_id(1)
    x = x_ref[...].astype(jnp.bfloat16)

    def chunk(j):
        g = wrefs[3 * j][...].astype(jnp.bfloat16)
        up = wrefs[3 * j + 1][...].astype(jnp.bfloat16)
        hg = lax.dot_general(x, g, (((1,), (1,)), ((), ())),
                             preferred_element_type=jnp.float32)
        hu = lax.dot_general(x, up, (((1,), (1,)), ((), ())),
                             preferred_element_type=jnp.float32)
        h = hg * jax.nn.sigmoid(hg) * hu
        d = wrefs[3 * j + 2][...].astype(jnp.bfloat16)
        return lax.dot_general(h.astype(jnp.bfloat16), d,
                               (((1,), (1,)), ((), ())),
                               preferred_element_type=jnp.float32)

    part = chunk(0)

    @pl.when(k == 0)
    def _():
        out_ref[...] = part

    @pl.when(k != 0)
    def _():
        out_ref[...] += part

    for j in range(1, NCH):
        @pl.when(NCH * k + j < KS)
        def _(j=j):
            out_ref[...] += chunk(j)


def _sw_spec(j):
    return pl.BlockSpec((BF, D), lambda t, k: (_cj(k, j, KS), 0))


def _sd_spec(j):
    return pl.BlockSpec((D, BF), lambda t, k: (0, _cj(k, j, KS)))


def _run_shared(x, sh_gate, sh_up, sh_down):
    kp = -(-KS // NCH)
    wspecs = []
    wargs = []
    for j in range(NCH):
        wspecs += [_sw_spec(j), _sw_spec(j), _sd_spec(j)]
        wargs += [sh_gate, sh_up, sh_down]
    return pl.pallas_call(
        _shared_body,
        grid=(TS, kp),
        in_specs=[pl.BlockSpec((BM, D), lambda t, k: (t, 0))] + wspecs,
        out_specs=pl.BlockSpec((BM, D), lambda t, k: (t, 0)),
        out_shape=jax.ShapeDtypeStruct((S, D), jnp.float32),
    )(x, *wargs)


def _final_body(ysh_ref, z0_ref, z1_ref, out_ref):
    out_ref[...] = ysh_ref[...] + z0_ref[...] + z1_ref[...]


def _run_final(ysh, z):
    return pl.pallas_call(
        _final_body,
        grid=(TS,),
        in_specs=[
            pl.BlockSpec((BM, D), lambda t: (t, 0)),
            pl.BlockSpec((BM, D), lambda t: (t, 0)),
            pl.BlockSpec((BM, D), lambda t: (TS + t, 0)),
        ],
        out_specs=pl.BlockSpec((BM, D), lambda t: (t, 0)),
        out_shape=jax.ShapeDtypeStruct((S, D), jnp.float32),
    )(ysh, z, z)


def kernel(hidden_states, gate_weight, w_gate, w_up, w_down,
           sh_gate, sh_up, sh_down):
    bsz, seq, h = hidden_states.shape
    x = hidden_states.reshape(-1, h)

    w0, w1, e0, e1 = _run_router(x, gate_weight)

    eflat = jnp.concatenate([e0, e1], axis=1).reshape(-1)
    wflat = jnp.concatenate([w0, w1], axis=1).reshape(-1)
    perm = jnp.argsort(eflat).astype(jnp.int32)
    sorted_eid = eflat[perm]
    sorted_tid = (perm // TOPK).astype(jnp.int32)
    sorted_w = wflat[perm]
    pos = jnp.zeros((S * TOPK,), jnp.int32).at[perm].set(
        jnp.arange(S * TOPK, dtype=jnp.int32))

    counts = jnp.zeros((E,), jnp.int32).at[eflat].add(1)
    offs = (jnp.cumsum(counts) - counts).astype(jnp.int32)
    ntile = (counts + BM - 1) // BM
    cumt = jnp.cumsum(ntile).astype(jnp.int32)
    ptile = (cumt - ntile).astype(jnp.int32)
    total_tiles = cumt[-1]

    rarr = jnp.arange(S * TOPK, dtype=jnp.int32)
    qpos = ptile[sorted_eid] * BM + (rarr - offs[sorted_eid])
    tidp = (jnp.arange(NPAD, dtype=jnp.int32) % S).at[qpos].set(sorted_tid)
    swp = jnp.zeros((NPAD,), jnp.float32).at[qpos].set(sorted_w)
    swp = swp.reshape(NPAD, 1)
    qslot = jnp.zeros((S * TOPK,), jnp.int32).at[perm].set(qpos)
    pp = qslot.reshape(S, TOPK).T.reshape(-1)

    uarr = jnp.arange(WU, dtype=jnp.int32)
    eu = jnp.minimum(jnp.searchsorted(cumt, uarr, side="right"),
                     E - 1).astype(jnp.int32)
    valid_u = uarr < total_tiles
    last_e = sorted_eid[-1]
    ue = jnp.where(valid_u, eu, last_e).astype(jnp.int32)
    ut = jnp.where(valid_u, uarr, total_tiles - 1).astype(jnp.int32)
    us = jnp.zeros((WU,), jnp.int32)
    uen = jnp.where(valid_u, BM, 0).astype(jnp.int32)
    uf = valid_u.astype(jnp.int32)

    ysh = _run_shared(x, sh_gate, sh_up, sh_down)

    xs = _make_sc_gather(S, NPAD, 24)(x, tidp)

    y = _run_grouped(xs, swp, w_gate, w_up, w_down, ue, ut, us, uen, uf)

    z = _make_sc_gather(NPAD, S * TOPK, 16)(y, pp)

    out = _run_final(ysh, z)
    return out.reshape(bsz, seq, h)
